# baseline (device time: 175478 ns/iter reference)
import jax
import jax.numpy as jnp
from jax import lax
from jax.experimental import pallas as pl
from jax.experimental.pallas import tpu as pltpu

N_DEV = 8
K_TILE = 2048
N_SLOTS = 4
N_BUF = 4


def kernel(x, w_mat):
    m_per, k_dim = x.shape
    _, n_total = w_mat.shape
    n_per = n_total // N_DEV
    n_k = k_dim // K_TILE
    n_tiles = N_DEV * n_k

    my_out = lax.axis_index("i")
    perm = (my_out + jnp.arange(N_DEV, dtype=jnp.int32)) % N_DEV

    def body(perm_ref, x_ref, w_ref, out_ref, acc_ref, wbuf_ref,
             send_sems, recv_sems, wsems, local_sem):
        j = pl.program_id(0)
        k = pl.program_id(1)
        my = lax.axis_index("i")
        slot = lax.rem(j, N_SLOTS)
        t = j * n_k + k

        def start_fetch(t2):
            jd = lax.div(t2, n_k)
            kd = lax.rem(t2, n_k)
            b = lax.rem(t2, N_BUF)
            col = perm_ref[jd]
            pltpu.make_async_copy(
                w_ref.at[pl.ds(kd * K_TILE, K_TILE),
                         pl.ds(col * n_per, n_per)],
                wbuf_ref.at[b],
                wsems.at[b],
            ).start()

        @pl.when((j == 0) & (k == 0))
        def _entry():
            barrier = pltpu.get_barrier_semaphore()
            for p in range(N_DEV):
                pl.semaphore_signal(
                    barrier, inc=1,
                    device_id=(p,), device_id_type=pl.DeviceIdType.MESH,
                )
            pl.semaphore_wait(barrier, N_DEV)
            for t2 in range(N_BUF):
                start_fetch(jnp.int32(t2))

        @pl.when((j >= N_SLOTS + 1) & (k == 0))
        def _reuse_wait():
            rdma = pltpu.make_async_remote_copy(
                src_ref=acc_ref.at[slot],
                dst_ref=out_ref.at[pl.ds(0, m_per)],
                send_sem=send_sems.at[slot],
                recv_sem=recv_sems.at[0],
                device_id=(0,),
                device_id_type=pl.DeviceIdType.MESH,
            )
            rdma.wait_send()

        b = lax.rem(t, N_BUF)
        pltpu.make_async_copy(
            w_ref.at[pl.ds(0, K_TILE), pl.ds(0, n_per)],
            wbuf_ref.at[b],
            wsems.at[b],
        ).wait()

        @pl.when(t + N_BUF < n_tiles)
        def _prefetch():
            start_fetch(t + N_BUF)

        prod = jnp.dot(
            x_ref[:, pl.ds(k * K_TILE, K_TILE)],
            wbuf_ref[b],
            preferred_element_type=jnp.float32,
        )

        @pl.when(k == 0)
        def _init():
            acc_ref[slot] = prod

        @pl.when(k != 0)
        def _accum():
            acc_ref[slot] += prod

        @pl.when(k == n_k - 1)
        def _emit():
            @pl.when(j == 0)
            def _local():
                pltpu.make_async_copy(
                    acc_ref.at[slot],
                    out_ref.at[pl.ds(my * m_per, m_per)],
                    local_sem,
                ).start()

            @pl.when(j != 0)
            def _send():
                target = perm_ref[j]
                rdma = pltpu.make_async_remote_copy(
                    src_ref=acc_ref.at[slot],
                    dst_ref=out_ref.at[pl.ds(my * m_per, m_per)],
                    send_sem=send_sems.at[slot],
                    recv_sem=recv_sems.at[my],
                    device_id=(target,),
                    device_id_type=pl.DeviceIdType.MESH,
                )
                rdma.start()

        @pl.when((j == N_DEV - 1) & (k == n_k - 1))
        def _drain():
            pltpu.make_async_copy(
                acc_ref.at[0],
                out_ref.at[pl.ds(my * m_per, m_per)],
                local_sem,
            ).wait()
            for p in range(N_DEV):
                @pl.when(p != my)
                def _wait_recv(p=p):
                    rdma = pltpu.make_async_remote_copy(
                        src_ref=acc_ref.at[0],
                        dst_ref=out_ref.at[pl.ds(p * m_per, m_per)],
                        send_sem=send_sems.at[0],
                        recv_sem=recv_sems.at[p],
                        device_id=(p,),
                        device_id_type=pl.DeviceIdType.MESH,
                    )
                    rdma.wait_recv()
            for s in range(N_SLOTS):
                rdma = pltpu.make_async_remote_copy(
                    src_ref=acc_ref.at[s],
                    dst_ref=out_ref.at[pl.ds(0, m_per)],
                    send_sem=send_sems.at[s],
                    recv_sem=recv_sems.at[0],
                    device_id=(0,),
                    device_id_type=pl.DeviceIdType.MESH,
                )
                rdma.wait_send()

    return pl.pallas_call(
        body,
        grid_spec=pltpu.PrefetchScalarGridSpec(
            num_scalar_prefetch=1,
            grid=(N_DEV, n_k),
            in_specs=[
                pl.BlockSpec(memory_space=pltpu.MemorySpace.VMEM),
                pl.BlockSpec(memory_space=pl.ANY),
            ],
            out_specs=pl.BlockSpec(memory_space=pl.ANY),
            scratch_shapes=[
                pltpu.VMEM((N_SLOTS, m_per, n_per), jnp.float32),
                pltpu.VMEM((N_BUF, K_TILE, n_per), jnp.float32),
                pltpu.SemaphoreType.DMA((N_SLOTS,)),
                pltpu.SemaphoreType.DMA((N_DEV,)),
                pltpu.SemaphoreType.DMA((N_BUF,)),
                pltpu.SemaphoreType.DMA,
            ],
        ),
        out_shape=jax.ShapeDtypeStruct((N_DEV * m_per, n_per), jnp.float32),
        compiler_params=pltpu.CompilerParams(
            dimension_semantics=("arbitrary", "arbitrary"),
            collective_id=0,
            vmem_limit_bytes=128 * 1024 * 1024,
        ),
    )(perm, x, w_mat)
